# Initial kernel scaffold; baseline (speedup 1.0000x reference)
import jax
import jax.numpy as jnp
from jax.experimental import pallas as pl


def kernel(x, router_w, router_b, w1, b1, w2, b2):
    raise NotImplementedError("write your pallas kernel here")


# speedup vs baseline: 3.0838x; 3.0838x over previous
import functools

import jax
import jax.numpy as jnp
from jax import lax
from jax.experimental import pallas as pl
from jax.experimental.pallas import tpu as pltpu

E = 8
D = 768
H = 192
M = 256


def _moe_block_kernel(x_ref, rw_ref, rb_ref, w1_ref, b1_ref, w2_ref, b2_ref,
                      out_ref):
    x = x_ref[...]
    logits = jnp.dot(x, rw_ref[...].T, preferred_element_type=jnp.float32)
    logits = logits + rb_ref[...]
    m = jnp.max(logits, axis=1, keepdims=True)
    iota_e = lax.broadcasted_iota(jnp.int32, (M, E), 1)
    eid = jnp.min(jnp.where(logits == m, iota_e, E), axis=1, keepdims=True)
    acc = jnp.zeros((M, D), dtype=jnp.float32)
    for e in range(E):
        h = jnp.dot(x, w1_ref[e], preferred_element_type=jnp.float32)
        h = jnp.maximum(h + b1_ref[e], 0.0)
        y = jnp.dot(h, w2_ref[e], preferred_element_type=jnp.float32)
        y = jnp.maximum(y + b2_ref[e], 0.0)
        acc = jnp.where(eid == e, y, acc)
    out_ref[...] = acc


def kernel(x, router_w, router_b, w1, b1, w2, b2):
    B, K, Dq = x.shape
    N = B * K
    x_flat = x.reshape(N, Dq)
    grid = (N // M,)
    out = pl.pallas_call(
        _moe_block_kernel,
        grid=grid,
        in_specs=[
            pl.BlockSpec((M, D), lambda i: (i, 0)),
            pl.BlockSpec((E, D), lambda i: (0, 0)),
            pl.BlockSpec((1, E), lambda i: (0, 0)),
            pl.BlockSpec((E, D, H), lambda i: (0, 0, 0)),
            pl.BlockSpec((E, H), lambda i: (0, 0)),
            pl.BlockSpec((E, H, D), lambda i: (0, 0, 0)),
            pl.BlockSpec((E, D), lambda i: (0, 0)),
        ],
        out_specs=pl.BlockSpec((M, D), lambda i: (i, 0)),
        out_shape=jax.ShapeDtypeStruct((N, D), jnp.float32),
    )(x_flat, router_w, router_b.reshape(1, E), w1, b1, w2, b2)
    return out.reshape(B, K, Dq)
